# Initial kernel scaffold; baseline (speedup 1.0000x reference)
import jax
import jax.numpy as jnp
from jax.experimental import pallas as pl


def kernel(xyz1, xyz2, nor1, nor2):
    raise NotImplementedError("write your pallas kernel here")


# speedup vs baseline: 1.6121x; 1.6121x over previous
import jax
import jax.numpy as jnp
from jax.experimental import pallas as pl

_LOG2E = 1.4426950408889634


def _dotT(a, b):
    return jax.lax.dot_general(
        a, b, (((1,), (1,)), ((), ())), preferred_element_type=jnp.float32
    )


def _varifold_tile(x1_ref, n1_ref, x2_ref, n2_ref, out_ref):
    x1 = x1_ref[0]
    n1 = n1_ref[0]
    x2 = x2_ref[0]
    n2 = n2_ref[0]

    T = x1.shape[0]
    N2 = x2.shape[0]
    ones_c = jnp.ones((T, 1), dtype=jnp.float32)
    x1sq = jnp.sum(x1 * x1, axis=1, keepdims=True)
    x2sq = jnp.sum(x2 * x2, axis=1, keepdims=True)

    lhs = jnp.concatenate([x1, ones_c, x1sq], axis=1)
    rhsT = jnp.concatenate(
        [(2.0 * _LOG2E) * x2, -_LOG2E * x2sq,
         jnp.full((N2, 1), -_LOG2E, dtype=jnp.float32)],
        axis=1,
    )

    arg = _dotT(lhs, rhsT)
    dotn = _dotT(n1, n2)

    s = jnp.exp2(arg) * (dotn * dotn)

    r = s.shape[0]
    while r > 8:
        r //= 2
        s = s[:r, :] + s[r:, :]
    c = s.shape[1]
    while c > 128:
        c //= 2
        s = s[:, :c] + s[:, c:]

    part = jnp.sum(s, keepdims=True)

    i = pl.program_id(1)

    @pl.when(i == 0)
    def _():
        out_ref[0] = part

    @pl.when(i != 0)
    def _():
        out_ref[0] += part


def kernel(xyz1, xyz2, nor1, nor2):
    B, N1, _ = xyz1.shape
    N2 = xyz2.shape[1]
    T = 2048

    out = pl.pallas_call(
        _varifold_tile,
        grid=(B, N1 // T),
        in_specs=[
            pl.BlockSpec((1, T, 3), lambda b, i: (b, i, 0)),
            pl.BlockSpec((1, T, 3), lambda b, i: (b, i, 0)),
            pl.BlockSpec((1, N2, 3), lambda b, i: (b, 0, 0)),
            pl.BlockSpec((1, N2, 3), lambda b, i: (b, 0, 0)),
        ],
        out_specs=pl.BlockSpec((1, 1, 1), lambda b, i: (b, 0, 0)),
        out_shape=jax.ShapeDtypeStruct((B, 1, 1), jnp.float32),
    )(xyz1, nor1, xyz2, nor2)
    return out[:, 0, 0]


# speedup vs baseline: 1.7664x; 1.0957x over previous
import jax
import jax.numpy as jnp
from jax.experimental import pallas as pl

_LOG2E = 1.4426950408889634


def _varifold_all(x1_ref, n1_ref, x2t_ref, n2t_ref, out_ref):
    B = x1_ref.shape[0]
    N2 = x2t_ref.shape[2]

    for b in range(B):
        x1 = x1_ref[b]
        n1 = n1_ref[b]
        x2 = x2t_ref[b]
        n2 = n2t_ref[b]

        T = x1.shape[0]
        ones_r = jnp.ones((T, 1), dtype=jnp.float32)
        x1sq = jnp.sum(x1 * x1, axis=1, keepdims=True)
        x2sq = jnp.sum(x2 * x2, axis=0, keepdims=True)

        lhs = jnp.concatenate([x1, ones_r, x1sq], axis=1)
        rhs = jnp.concatenate(
            [(2.0 * _LOG2E) * x2, -_LOG2E * x2sq,
             jnp.full((1, N2), -_LOG2E, dtype=jnp.float32)],
            axis=0,
        )

        arg = jnp.dot(lhs, rhs, preferred_element_type=jnp.float32)
        dotn = jnp.dot(n1, n2, preferred_element_type=jnp.float32)

        s = jnp.exp2(arg) * (dotn * dotn)

        r = s.shape[0]
        while r > 8:
            r //= 2
            s = s[:r, :] + s[r:, :]
        c = s.shape[1]
        while c > 128:
            c //= 2
            s = s[:, :c] + s[:, c:]

        out_ref[b] = jnp.sum(s, keepdims=True)


def kernel(xyz1, xyz2, nor1, nor2):
    B, N1, _ = xyz1.shape
    N2 = xyz2.shape[1]

    x2t = jnp.swapaxes(xyz2, 1, 2)
    n2t = jnp.swapaxes(nor2, 1, 2)

    out = pl.pallas_call(
        _varifold_all,
        out_shape=jax.ShapeDtypeStruct((B, 1, 1), jnp.float32),
    )(xyz1, nor1, x2t, n2t)
    return out[:, 0, 0]


# speedup vs baseline: 2.2433x; 1.2700x over previous
import jax
import jax.numpy as jnp
from jax.experimental import pallas as pl

_LOG2E = 1.4426950408889634


def _dot_tl(aT, b):
    return jax.lax.dot_general(
        aT, b, (((0,), (0,)), ((), ())), preferred_element_type=jnp.float32
    )


def _varifold_tile(x1t_ref, n1t_ref, x2t_ref, n2t_ref, out_ref):
    x1t = x1t_ref[0]
    n1t = n1t_ref[0]
    x2 = x2t_ref[0]
    n2 = n2t_ref[0]

    T = x1t.shape[1]
    N2 = x2.shape[1]
    x1sqT = jnp.sum(x1t * x1t, axis=0, keepdims=True)
    x2sq = jnp.sum(x2 * x2, axis=0, keepdims=True)

    lhsT = jnp.concatenate(
        [x1t, jnp.ones((1, T), dtype=jnp.float32), x1sqT], axis=0
    )
    rhs = jnp.concatenate(
        [(2.0 * _LOG2E) * x2, -_LOG2E * x2sq,
         jnp.full((1, N2), -_LOG2E, dtype=jnp.float32)],
        axis=0,
    )

    arg = _dot_tl(lhsT, rhs)
    dotn = _dot_tl(n1t, n2)

    s = jnp.exp2(arg) * (dotn * dotn)

    r = s.shape[0]
    while r > 8:
        r //= 2
        s = s[:r, :] + s[r:, :]
    c = s.shape[1]
    while c > 128:
        c //= 2
        s = s[:, :c] + s[:, c:]

    part = jnp.sum(s, keepdims=True)

    i = pl.program_id(1)

    @pl.when(i == 0)
    def _():
        out_ref[0] = part

    @pl.when(i != 0)
    def _():
        out_ref[0] += part


def kernel(xyz1, xyz2, nor1, nor2):
    B, N1, _ = xyz1.shape
    N2 = xyz2.shape[1]
    T = 2048

    x1t = jnp.swapaxes(xyz1, 1, 2)
    n1t = jnp.swapaxes(nor1, 1, 2)
    x2t = jnp.swapaxes(xyz2, 1, 2)
    n2t = jnp.swapaxes(nor2, 1, 2)

    out = pl.pallas_call(
        _varifold_tile,
        grid=(B, N1 // T),
        in_specs=[
            pl.BlockSpec((1, 3, T), lambda b, i: (b, 0, i)),
            pl.BlockSpec((1, 3, T), lambda b, i: (b, 0, i)),
            pl.BlockSpec((1, 3, N2), lambda b, i: (b, 0, 0)),
            pl.BlockSpec((1, 3, N2), lambda b, i: (b, 0, 0)),
        ],
        out_specs=pl.BlockSpec((1, 1, 1), lambda b, i: (b, 0, 0)),
        out_shape=jax.ShapeDtypeStruct((B, 1, 1), jnp.float32),
    )(x1t, n1t, x2t, n2t)
    return out[:, 0, 0]


# speedup vs baseline: 2.2649x; 1.0096x over previous
import jax
import jax.numpy as jnp
from jax.experimental import pallas as pl

_LOG2E = 1.4426950408889634


def _dot_tl(aT, b):
    return jax.lax.dot_general(
        aT, b, (((0,), (0,)), ((), ())), preferred_element_type=jnp.float32
    )


def _varifold_batch(x1t_ref, n1t_ref, x2t_ref, n2t_ref, out_ref):
    x1t = x1t_ref[0]
    n1t = n1t_ref[0]
    x2 = x2t_ref[0]
    n2 = n2t_ref[0]

    T = x1t.shape[1]
    N2 = x2.shape[1]
    x1sqT = jnp.sum(x1t * x1t, axis=0, keepdims=True)
    x2sq = jnp.sum(x2 * x2, axis=0, keepdims=True)

    lhsT = jnp.concatenate(
        [x1t, jnp.ones((1, T), dtype=jnp.float32), x1sqT], axis=0
    )
    rhs = jnp.concatenate(
        [(2.0 * _LOG2E) * x2, -_LOG2E * x2sq,
         jnp.full((1, N2), -_LOG2E, dtype=jnp.float32)],
        axis=0,
    )

    arg = _dot_tl(lhsT, rhs)
    dotn = _dot_tl(n1t, n2)

    s = jnp.exp2(arg) * (dotn * dotn)

    r = s.shape[0]
    while r > 8:
        r //= 2
        s = s[:r, :] + s[r:, :]
    c = s.shape[1]
    while c > 128:
        c //= 2
        s = s[:, :c] + s[:, c:]

    out_ref[0] = jnp.sum(s, keepdims=True)


def kernel(xyz1, xyz2, nor1, nor2):
    B, N1, _ = xyz1.shape
    N2 = xyz2.shape[1]

    x1t = jnp.swapaxes(xyz1, 1, 2)
    n1t = jnp.swapaxes(nor1, 1, 2)
    x2t = jnp.swapaxes(xyz2, 1, 2)
    n2t = jnp.swapaxes(nor2, 1, 2)

    out = pl.pallas_call(
        _varifold_batch,
        grid=(B,),
        in_specs=[
            pl.BlockSpec((1, 3, N1), lambda b: (b, 0, 0)),
            pl.BlockSpec((1, 3, N1), lambda b: (b, 0, 0)),
            pl.BlockSpec((1, 3, N2), lambda b: (b, 0, 0)),
            pl.BlockSpec((1, 3, N2), lambda b: (b, 0, 0)),
        ],
        out_specs=pl.BlockSpec((1, 1, 1), lambda b: (b, 0, 0)),
        out_shape=jax.ShapeDtypeStruct((B, 1, 1), jnp.float32),
    )(x1t, n1t, x2t, n2t)
    return out[:, 0, 0]
